# Initial kernel scaffold; baseline (speedup 1.0000x reference)
import jax
import jax.numpy as jnp
from jax.experimental import pallas as pl


def kernel(x, edge_index, batch, eps, W1, W2, Wmu, Wlv, Wd):
    raise NotImplementedError("write your pallas kernel here")


# speedup vs baseline: 3.5928x; 3.5928x over previous
import functools

import jax
import jax.numpy as jnp
from jax import lax
from jax.experimental import pallas as pl
from jax.experimental.pallas import tpu as pltpu
from jax.experimental.pallas import tpu_sc as plsc

N = 10000
E = 320000
D = 128
H = 256
LD = 64
NUM_SEGMENTS = 256.0

NC = 2
NS = 16
LANES = 16

LOG2PI = 1.8378770664093453


def _sc_mesh():
    return plsc.VectorSubcoreMesh(
        core_axis_name="c", subcore_axis_name="s", num_cores=NC, num_subcores=NS
    )


_CH_A = 80
_EPC = E // NC
_EPT = _EPC // NS
_NCH_A = _EPT // _CH_A
_RSTRIDE = 624
_RWIN = 640


def _agg_body(src_hbm, dst_hbm, x_hbm, zeros_hbm, out_hbm, sidx, didx, rows, acc, sem):
    c = lax.axis_index("c")
    s = lax.axis_index("s")
    pltpu.sync_copy(
        zeros_hbm.at[pl.ds(s * _RSTRIDE, _RWIN)], acc.at[pl.ds(s * _RSTRIDE, _RWIN)]
    )
    plsc.subcore_barrier()
    base0 = c * _EPC + s * _EPT

    def chunk(i, carry):
        base = base0 + i * _CH_A
        pltpu.sync_copy(src_hbm.at[pl.ds(base, _CH_A)], sidx)
        pltpu.sync_copy(dst_hbm.at[pl.ds(base, _CH_A)], didx)
        pltpu.async_copy(x_hbm.at[sidx], rows, sem).wait()
        pltpu.sync_copy(rows, acc.at[didx], add=True)
        return carry

    lax.fori_loop(0, _NCH_A, chunk, 0)
    plsc.subcore_barrier()
    pltpu.sync_copy(
        acc.at[pl.ds(s * _RSTRIDE, _RWIN)], out_hbm.at[c, pl.ds(s * _RSTRIDE, _RWIN)]
    )


@functools.lru_cache(maxsize=None)
def _agg_call():
    return functools.partial(
        pl.kernel,
        out_type=jax.ShapeDtypeStruct((NC, N, D), jnp.float32),
        mesh=_sc_mesh(),
        compiler_params=pltpu.CompilerParams(needs_layout_passes=False),
        scratch_types=[
            pltpu.VMEM((_CH_A,), jnp.int32),
            pltpu.VMEM((_CH_A,), jnp.int32),
            pltpu.VMEM((_CH_A, D), jnp.float32),
            pltpu.VMEM_SHARED((N, D), jnp.float32),
            pltpu.SemaphoreType.DMA,
        ],
    )(_agg_body)


_RB = 2000
_NB = N // _RB


def _dense_body(p0, p1, x, eps, w1, w2, wmu, wlv, wd, z_out, kl_out, nlp_out):
    i = pl.program_id(0)
    agg = p0[...] + p1[...]
    h = jnp.maximum(
        jnp.dot(agg, w1[...], preferred_element_type=jnp.float32)
        + jnp.dot(x[...], w2[...], preferred_element_type=jnp.float32),
        0.0,
    )
    mu = jnp.dot(h, wmu[...], preferred_element_type=jnp.float32)
    lv = jnp.clip(jnp.dot(h, wlv[...], preferred_element_type=jnp.float32), -8.0, 8.0)
    s2 = jnp.exp(lv)
    z = mu + jnp.exp(0.5 * lv) * eps[...]
    z_out[...] = z
    klb = 0.5 * jnp.sum(mu * mu + s2 - 1.0 - lv)
    xr = jnp.dot(z, wd[...], preferred_element_type=jnp.float32)
    nlb = -0.5 * jnp.sum((x[...] - xr) ** 2) - 0.5 * _RB * D * LOG2PI

    @pl.when(i == 0)
    def _():
        kl_out[0, 0] = klb
        nlp_out[0, 0] = nlb

    @pl.when(i != 0)
    def _():
        kl_out[0, 0] += klb
        nlp_out[0, 0] += nlb


def _dense_call(p0, p1, x, eps, w1, w2, wmu, wlv, wd):
    full = lambda shape: pl.BlockSpec(shape, lambda i: (0, 0))
    blk = lambda shape: pl.BlockSpec(shape, lambda i: (i, 0))
    scalar = pl.BlockSpec((1, 1), lambda i: (0, 0), memory_space=pltpu.SMEM)
    return pl.pallas_call(
        _dense_body,
        grid=(_NB,),
        in_specs=[
            blk((_RB, D)), blk((_RB, D)), blk((_RB, D)), blk((_RB, LD)),
            full((D, H)), full((D, H)), full((H, LD)), full((H, LD)), full((LD, D)),
        ],
        out_specs=[blk((_RB, LD)), scalar, scalar],
        out_shape=[
            jax.ShapeDtypeStruct((N, LD), jnp.float32),
            jax.ShapeDtypeStruct((1, 1), jnp.float32),
            jax.ShapeDtypeStruct((1, 1), jnp.float32),
        ],
    )(p0, p1, x, eps, w1, w2, wmu, wlv, wd)


_CH_C = 80
_NCH_C = _EPT // _CH_C


def _edge_body(src_hbm, dst_hbm, z_hbm, logit_hbm, sidx, didx, zs, zd, lbuf, sem):
    c = lax.axis_index("c")
    s = lax.axis_index("s")
    base0 = c * _EPC + s * _EPT

    def chunk(i, carry):
        base = base0 + i * _CH_C
        pltpu.sync_copy(src_hbm.at[pl.ds(base, _CH_C)], sidx)
        pltpu.sync_copy(dst_hbm.at[pl.ds(base, _CH_C)], didx)
        pltpu.async_copy(z_hbm.at[sidx], zs, sem).wait()
        pltpu.async_copy(z_hbm.at[didx], zd, sem).wait()

        def egroup(g, carry2):
            rowi = g * LANES + lax.iota(jnp.int32, LANES)

            def dcol(d, acc):
                coli = jnp.full((LANES,), d, jnp.int32)
                a = plsc.load_gather(zs, [rowi, coli])
                b = plsc.load_gather(zd, [rowi, coli])
                return acc + a * b

            acc = lax.fori_loop(0, LD, dcol, jnp.zeros((LANES,), jnp.float32))
            lbuf[pl.ds(g * LANES, LANES)] = acc
            return carry2

        lax.fori_loop(0, _CH_C // LANES, egroup, 0)
        pltpu.sync_copy(lbuf, logit_hbm.at[pl.ds(base, _CH_C)])
        return carry

    lax.fori_loop(0, _NCH_C, chunk, 0)


@functools.lru_cache(maxsize=None)
def _edge_call():
    return functools.partial(
        pl.kernel,
        out_type=jax.ShapeDtypeStruct((E,), jnp.float32),
        mesh=_sc_mesh(),
        compiler_params=pltpu.CompilerParams(
            needs_layout_passes=False, use_tc_tiling_on_sc=False
        ),
        scratch_types=[
            pltpu.VMEM((_CH_C,), jnp.int32),
            pltpu.VMEM((_CH_C,), jnp.int32),
            pltpu.VMEM((_CH_C, LD), jnp.float32),
            pltpu.VMEM((_CH_C, LD), jnp.float32),
            pltpu.VMEM((_CH_C,), jnp.float32),
            pltpu.SemaphoreType.DMA,
        ],
    )(_edge_body)


def _tail_body(l_ref, kl_ref, nlp_ref, out_ref):
    t = l_ref[...]
    elp = jnp.sum(jnp.minimum(t, 0.0) - jnp.log1p(jnp.exp(-jnp.abs(t))))
    out_ref[0, 0] = -((nlp_ref[0, 0] + elp - kl_ref[0, 0]) / NUM_SEGMENTS)


def _tail_call(logits2d, kl_s, nlp_s):
    scalar = pl.BlockSpec(memory_space=pltpu.SMEM)
    return pl.pallas_call(
        _tail_body,
        in_specs=[pl.BlockSpec(logits2d.shape, lambda: (0, 0)), scalar, scalar],
        out_specs=scalar,
        out_shape=jax.ShapeDtypeStruct((1, 1), jnp.float32),
    )(logits2d, kl_s, nlp_s)


def kernel(x, edge_index, batch, eps, W1, W2, Wmu, Wlv, Wd):
    del batch
    src = edge_index[0]
    dst = edge_index[1]
    zeros = jnp.zeros((N, D), jnp.float32)
    parts = _agg_call()(src, dst, x, zeros)
    z, kl_s, nlp_s = _dense_call(parts[0], parts[1], x, eps, W1, W2, Wmu, Wlv, Wd)
    logits = _edge_call()(src, dst, z)
    out = _tail_call(logits.reshape(E // D, D), kl_s, nlp_s)
    return out[0, 0]
